# baseline (speedup 1.0000x reference)
import functools

import jax
import jax.numpy as jnp
from jax import lax
from jax.experimental import pallas as pl
from jax.experimental.pallas import tpu as pltpu
from jax.experimental.pallas import tpu_sc as plsc

VOCAB = 1000000
EMBED = 64
B = 16384
C = 4
K = 20

NC = 2
NS = 16
NW = NC * NS
BPW = B // NW
CH = 64
NCHUNK = BPW // CH
NQ = 4
GMAX = 128
GI = 4


def _sc_body(tgt_hbm, ctx_hbm, noi_hbm, ein_hbm, eout_hbm,
             pos_hbm, neg_hbm,
             tgt_idx, ctx_idx, noi_idx,
             tgt_rows, ctx_rows, noi_rows,
             pos_all, neg_all, sem):
    wid = lax.axis_index("s") * NC + lax.axis_index("c")
    base = wid * BPW
    lanes = lax.iota(jnp.int32, 16)

    def chunk_body(g, _):
        cb = base + g * CH
        ob = g * CH

        pltpu.sync_copy(tgt_hbm.at[pl.ds(cb, CH)], tgt_idx)
        pltpu.sync_copy(ctx_hbm.at[pl.ds(cb * C, CH * C)], ctx_idx)
        pltpu.sync_copy(noi_hbm.at[pl.ds(cb * K, CH * K)], noi_idx)

        cps = []
        for q in range(0, CH, GMAX):
            n = min(GMAX, CH - q)
            cps.append(pltpu.async_copy(
                ein_hbm.at[tgt_idx.at[pl.ds(q, n)]],
                tgt_rows.at[pl.ds(q, n)], sem))
        for q in range(0, CH * C, GMAX):
            n = min(GMAX, CH * C - q)
            cps.append(pltpu.async_copy(
                eout_hbm.at[ctx_idx.at[pl.ds(q, n)]],
                ctx_rows.at[pl.ds(q, n)], sem))
        for q in range(0, CH * K, GMAX):
            n = min(GMAX, CH * K - q)
            cps.append(pltpu.async_copy(
                eout_hbm.at[noi_idx.at[pl.ds(q, n)]],
                noi_rows.at[pl.ds(q, n)], sem))
        for cp in cps:
            cp.wait()

        def group_body(g4, _):
            i0 = g4 * GI
            t = [[tgt_rows[i0 + ii, pl.ds(16 * q, 16)] for q in range(NQ)]
                 for ii in range(GI)]

            def score(rows_ref, rowbase, r, per_item):
                it = r // per_item
                rw = rowbase + r
                rv = [rows_ref[rw, pl.ds(16 * q, 16)] for q in range(NQ)]
                p = (t[it][0] * rv[0] + t[it][1] * rv[1]) + \
                    (t[it][2] * rv[2] + t[it][3] * rv[3])
                return jnp.full((16,), jnp.sum(p), jnp.float32)

            acc = jnp.zeros((16,), jnp.float32)
            for r in range(GI * C):
                acc = jnp.where(lanes == r, score(ctx_rows, i0 * C, r, C), acc)
            pos_all[pl.ds((ob + i0) * C, 16)] = acc

            for a in range(GI * K // 16):
                acc = jnp.zeros((16,), jnp.float32)
                for rr in range(16):
                    r = a * 16 + rr
                    acc = jnp.where(lanes == rr,
                                    score(noi_rows, i0 * K, r, K), acc)
                neg_all[pl.ds((ob + i0) * K + a * 16, 16)] = acc
            return 0

        lax.fori_loop(0, CH // GI, group_body, 0)
        return 0

    lax.fori_loop(0, NCHUNK, chunk_body, 0)

    pltpu.sync_copy(pos_all, pos_hbm.at[pl.ds(base * C, BPW * C)])
    pltpu.sync_copy(neg_all, neg_hbm.at[pl.ds(base * K, BPW * K)])


@jax.jit
def _sc_call(tgt, ctx_flat, noi_flat, ein, eout):
    mesh = plsc.VectorSubcoreMesh(core_axis_name="c", subcore_axis_name="s")
    kfn = functools.partial(
        pl.kernel,
        mesh=mesh,
        compiler_params=pltpu.CompilerParams(
            needs_layout_passes=False, use_tc_tiling_on_sc=False),
        out_type=(
            jax.ShapeDtypeStruct((B * C,), jnp.float32),
            jax.ShapeDtypeStruct((B * K,), jnp.float32),
        ),
        scratch_types=[
            pltpu.VMEM((CH,), jnp.int32),
            pltpu.VMEM((CH * C,), jnp.int32),
            pltpu.VMEM((CH * K,), jnp.int32),
            pltpu.VMEM((CH, EMBED), jnp.float32),
            pltpu.VMEM((CH * C, EMBED), jnp.float32),
            pltpu.VMEM((CH * K, EMBED), jnp.float32),
            pltpu.VMEM((BPW * C,), jnp.float32),
            pltpu.VMEM((BPW * K,), jnp.float32),
            pltpu.SemaphoreType.DMA,
        ],
    )(_sc_body)
    return kfn(tgt, ctx_flat, noi_flat, ein, eout)


def kernel(target, context, noise, emb_in, emb_out):
    tgt = target.astype(jnp.int32)
    ctx_flat = context.astype(jnp.int32).reshape(-1)
    noi_flat = noise.astype(jnp.int32).reshape(-1)
    pos_flat, neg_flat = _sc_call(tgt, ctx_flat, noi_flat, emb_in, emb_out)
    return pos_flat.reshape(B, C), neg_flat.reshape(B, K)
